# baseline (speedup 1.0000x reference)
import functools

import jax
import jax.numpy as jnp
from jax import lax
from jax.experimental import pallas as pl
from jax.experimental.pallas import tpu as pltpu
from jax.experimental.pallas import tpu_sc as plsc


def _make_sc_reverse(B, Z):
    info = plsc.get_sparse_core_info()
    NC, NS, L = info.num_cores, info.num_subcores, info.num_lanes
    NW = NC * NS
    rows_per_w = B // NW
    R = 8
    nblocks = rows_per_w // R
    G = Z // L

    mesh = plsc.VectorSubcoreMesh(core_axis_name="c", subcore_axis_name="s")

    @functools.partial(
        pl.kernel,
        mesh=mesh,
        out_type=jax.ShapeDtypeStruct((B, Z), jnp.float32),
        scratch_types=[
            pltpu.VMEM((R, Z), jnp.float32),
            pltpu.VMEM((R, Z), jnp.float32),
        ],
    )
    def k(z_hbm, out_hbm, in_v, out_v):
        wid = lax.axis_index("s") * NC + lax.axis_index("c")
        base = wid * rows_per_w

        def block(b, carry):
            r0 = base + b * R
            pltpu.sync_copy(z_hbm.at[pl.ds(r0, R)], in_v)

            def row(r, carry2):
                def gran(j, carry3):
                    v = in_v[r, pl.ds(j * L, L)]
                    out_v[r, pl.ds((G - 1 - j) * L, L)] = jnp.flip(v)
                    return carry3

                return lax.fori_loop(0, G, gran, carry2, unroll=8)

            lax.fori_loop(0, R, row, 0)
            pltpu.sync_copy(out_v, out_hbm.at[pl.ds(r0, R)])
            return carry

        lax.fori_loop(0, nblocks, block, 0)

    return k


def kernel(z):
    B, Z = z.shape
    return _make_sc_reverse(B, Z)(z)


# speedup vs baseline: 4.1886x; 4.1886x over previous
import functools

import jax
import jax.numpy as jnp
from jax import lax
from jax.experimental import pallas as pl
from jax.experimental.pallas import tpu as pltpu
from jax.experimental.pallas import tpu_sc as plsc


def _make_sc_reverse(B, Z):
    info = plsc.get_sparse_core_info()
    NC, NS, L = info.num_cores, info.num_subcores, info.num_lanes
    NW = NC * NS
    rows_per_w = B // NW
    R = 4
    nblocks = rows_per_w // R
    G = Z // L
    H = G // 2

    mesh = plsc.VectorSubcoreMesh(core_axis_name="c", subcore_axis_name="s")

    def _reverse_block(in_v, out_v):
        @plsc.parallel_loop(0, R * H, unroll=8)
        def _(k):
            r = lax.shift_right_logical(k, 7)
            jj = lax.bitwise_and(k, H - 1)
            ja = jj * L
            jb = (G - 1 - jj) * L
            va = in_v[r, pl.ds(ja, L)]
            vb = in_v[r, pl.ds(jb, L)]
            out_v[r, pl.ds(jb, L)] = jnp.flip(va)
            out_v[r, pl.ds(ja, L)] = jnp.flip(vb)

    @functools.partial(
        pl.kernel,
        mesh=mesh,
        out_type=jax.ShapeDtypeStruct((B, Z), jnp.float32),
        scratch_types=[
            pltpu.VMEM((R, Z), jnp.float32),
            pltpu.VMEM((R, Z), jnp.float32),
            pltpu.VMEM((R, Z), jnp.float32),
            pltpu.VMEM((R, Z), jnp.float32),
            pltpu.SemaphoreType.DMA,
            pltpu.SemaphoreType.DMA,
            pltpu.SemaphoreType.DMA,
            pltpu.SemaphoreType.DMA,
        ],
    )
    def k(z_hbm, out_hbm, in0, in1, out0, out1, is0, is1, os0, os1):
        wid = lax.axis_index("s") * NC + lax.axis_index("c")
        base = wid * rows_per_w
        ins = (in0, in1)
        outs = (out0, out1)
        isems = (is0, is1)
        osems = (os0, os1)

        def src_at(b):
            return z_hbm.at[pl.ds(base + b * R, R)]

        def dst_at(b):
            return out_hbm.at[pl.ds(base + b * R, R)]

        pltpu.async_copy(src_at(0), ins[0], isems[0])
        pltpu.async_copy(src_at(1), ins[1], isems[1])

        def body(i, carry):
            for p in range(2):
                b = i * 2 + p
                pltpu.make_async_copy(src_at(b), ins[p], isems[p]).wait()

                @pl.when(b >= 2)
                def _():
                    pltpu.make_async_copy(outs[p], dst_at(b - 2), osems[p]).wait()

                _reverse_block(ins[p], outs[p])
                pltpu.async_copy(outs[p], dst_at(b), osems[p])

                @pl.when(b + 2 < nblocks)
                def _():
                    pltpu.async_copy(src_at(b + 2), ins[p], isems[p])

            return carry

        lax.fori_loop(0, nblocks // 2, body, 0)
        pltpu.make_async_copy(outs[0], dst_at(nblocks - 2), osems[0]).wait()
        pltpu.make_async_copy(outs[1], dst_at(nblocks - 1), osems[1]).wait()

    return k


def kernel(z):
    B, Z = z.shape
    return _make_sc_reverse(B, Z)(z)


# speedup vs baseline: 4.1921x; 1.0008x over previous
import functools

import jax
import jax.numpy as jnp
from jax import lax
from jax.experimental import pallas as pl
from jax.experimental.pallas import tpu as pltpu
from jax.experimental.pallas import tpu_sc as plsc


def _make_sc_reverse(B, Z):
    info = plsc.get_sparse_core_info()
    NC, NS, L = info.num_cores, info.num_subcores, info.num_lanes
    NW = NC * NS
    rows_per_w = B // NW
    R = 4
    nblocks = rows_per_w // R
    G = Z // L
    H = G // 2

    mesh = plsc.VectorSubcoreMesh(core_axis_name="c", subcore_axis_name="s")

    def _reverse_block(in_v, out_v):
        @plsc.parallel_loop(0, R * H, unroll=16)
        def _(k):
            r = lax.shift_right_logical(k, 7)
            jj = lax.bitwise_and(k, H - 1)
            ja = jj * L
            jb = (G - 1 - jj) * L
            va = in_v[r, pl.ds(ja, L)]
            vb = in_v[r, pl.ds(jb, L)]
            out_v[r, pl.ds(jb, L)] = jnp.flip(va)
            out_v[r, pl.ds(ja, L)] = jnp.flip(vb)

    @functools.partial(
        pl.kernel,
        mesh=mesh,
        out_type=jax.ShapeDtypeStruct((B, Z), jnp.float32),
        scratch_types=[
            pltpu.VMEM((R, Z), jnp.float32),
            pltpu.VMEM((R, Z), jnp.float32),
            pltpu.VMEM((R, Z), jnp.float32),
            pltpu.VMEM((R, Z), jnp.float32),
            pltpu.SemaphoreType.DMA,
            pltpu.SemaphoreType.DMA,
            pltpu.SemaphoreType.DMA,
            pltpu.SemaphoreType.DMA,
        ],
    )
    def k(z_hbm, out_hbm, in0, in1, out0, out1, is0, is1, os0, os1):
        wid = lax.axis_index("s") * NC + lax.axis_index("c")
        base = wid * rows_per_w
        ins = (in0, in1)
        outs = (out0, out1)
        isems = (is0, is1)
        osems = (os0, os1)

        def src_at(b):
            return z_hbm.at[pl.ds(base + b * R, R)]

        def dst_at(b):
            return out_hbm.at[pl.ds(base + b * R, R)]

        pltpu.async_copy(src_at(0), ins[0], isems[0])
        pltpu.async_copy(src_at(1), ins[1], isems[1])

        def body(i, carry):
            for p in range(2):
                b = i * 2 + p
                pltpu.make_async_copy(src_at(b), ins[p], isems[p]).wait()

                @pl.when(b >= 2)
                def _():
                    pltpu.make_async_copy(outs[p], dst_at(b - 2), osems[p]).wait()

                _reverse_block(ins[p], outs[p])
                pltpu.async_copy(outs[p], dst_at(b), osems[p])

                @pl.when(b + 2 < nblocks)
                def _():
                    pltpu.async_copy(src_at(b + 2), ins[p], isems[p])

            return carry

        lax.fori_loop(0, nblocks // 2, body, 0)
        pltpu.make_async_copy(outs[0], dst_at(nblocks - 2), osems[0]).wait()
        pltpu.make_async_copy(outs[1], dst_at(nblocks - 1), osems[1]).wait()

    return k


def kernel(z):
    B, Z = z.shape
    return _make_sc_reverse(B, Z)(z)


# speedup vs baseline: 4.2121x; 1.0048x over previous
import functools

import jax
import jax.numpy as jnp
from jax import lax
from jax.experimental import pallas as pl
from jax.experimental.pallas import tpu as pltpu
from jax.experimental.pallas import tpu_sc as plsc


def _make_sc_reverse(B, Z):
    info = plsc.get_sparse_core_info()
    NC, NS, L = info.num_cores, info.num_subcores, info.num_lanes
    NW = NC * NS
    rows_per_w = B // NW
    R = 8
    nblocks = rows_per_w // R
    G = Z // L
    H = G // 2
    NBUF = 3

    mesh = plsc.VectorSubcoreMesh(core_axis_name="c", subcore_axis_name="s")

    def _reverse_inplace(buf):
        @plsc.parallel_loop(0, R * H, unroll=8)
        def _(k):
            r = lax.shift_right_logical(k, 7)
            jj = lax.bitwise_and(k, H - 1)
            ja = jj * L
            jb = (G - 1 - jj) * L
            va = buf[r, pl.ds(ja, L)]
            vb = buf[r, pl.ds(jb, L)]
            buf[r, pl.ds(jb, L)] = jnp.flip(va)
            buf[r, pl.ds(ja, L)] = jnp.flip(vb)

    @functools.partial(
        pl.kernel,
        mesh=mesh,
        out_type=jax.ShapeDtypeStruct((B, Z), jnp.float32),
        scratch_types=[
            pltpu.VMEM((R, Z), jnp.float32),
            pltpu.VMEM((R, Z), jnp.float32),
            pltpu.VMEM((R, Z), jnp.float32),
            pltpu.SemaphoreType.DMA,
            pltpu.SemaphoreType.DMA,
            pltpu.SemaphoreType.DMA,
            pltpu.SemaphoreType.DMA,
            pltpu.SemaphoreType.DMA,
            pltpu.SemaphoreType.DMA,
        ],
    )
    def k(z_hbm, out_hbm, b0, b1, b2, is0, is1, is2, os0, os1, os2):
        wid = lax.axis_index("s") * NC + lax.axis_index("c")
        base = wid * rows_per_w
        bufs = (b0, b1, b2)
        isems = (is0, is1, is2)
        osems = (os0, os1, os2)

        def src_at(b):
            return z_hbm.at[pl.ds(base + b * R, R)]

        def dst_at(b):
            return out_hbm.at[pl.ds(base + b * R, R)]

        pltpu.async_copy(src_at(0), bufs[0], isems[0])
        pltpu.async_copy(src_at(1), bufs[1], isems[1])

        def body(i, carry):
            for p in range(NBUF):
                b = i * NBUF + p
                pltpu.make_async_copy(src_at(b), bufs[p], isems[p]).wait()
                _reverse_inplace(bufs[p])
                pltpu.async_copy(bufs[p], dst_at(b), osems[p])

                pn = (p + 2) % NBUF

                @pl.when(b + 2 < nblocks)
                def _():
                    @pl.when(b >= 1)
                    def _():
                        pltpu.make_async_copy(
                            bufs[pn], dst_at(b - 1), osems[pn]
                        ).wait()

                    pltpu.async_copy(src_at(b + 2), bufs[pn], isems[pn])

            return carry

        lax.fori_loop(0, nblocks // NBUF, body, 0)
        for b in range(nblocks - nblocks % NBUF, nblocks):
            p = b % NBUF
            pltpu.make_async_copy(src_at(b), bufs[p], isems[p]).wait()
            _reverse_inplace(bufs[p])
            pltpu.async_copy(bufs[p], dst_at(b), osems[p])
        for b in range(nblocks - NBUF, nblocks):
            pltpu.make_async_copy(bufs[b % NBUF], dst_at(b),
                                  osems[b % NBUF]).wait()

    return k


def kernel(z):
    B, Z = z.shape
    return _make_sc_reverse(B, Z)(z)
